# baseline (speedup 1.0000x reference)
import jax
import jax.numpy as jnp
from jax import lax
from jax.experimental import pallas as pl
from jax.experimental.pallas import tpu as pltpu

_EPS2 = 1e-24


def _silu(y):
    return y * jax.nn.sigmoid(y)


def _conv_norm_silu(y_ref, wp_ref, wh_ref, b_ref, g_ref, ho, wo, ci, co,
                    pad_edges):
    m = ho * wo
    c2 = 2 * ci
    acc = jnp.zeros((m, co), jnp.float32)
    for kh in range(4):
        base, par = kh // 2, kh % 2
        xs = y_ref[base:base + ho, par, 1:1 + wo, :].reshape(m, c2)
        acc = acc + jnp.dot(xs, wp_ref[kh],
                            preferred_element_type=jnp.float32)
        if pad_edges:
            x0 = y_ref[base:base + ho, par, 0:wo, :].reshape(m, c2)
            x2 = y_ref[base:base + ho, par, 2:2 + wo, :].reshape(m, c2)
        else:
            x0 = y_ref[base:base + ho, par, 0:wo, ci:].reshape(m, ci)
            x2 = y_ref[base:base + ho, par, 2:2 + wo, :ci].reshape(m, ci)
        acc = acc + jnp.dot(x0, wh_ref[2 * kh],
                            preferred_element_type=jnp.float32)
        acc = acc + jnp.dot(x2, wh_ref[2 * kh + 1],
                            preferred_element_type=jnp.float32)
    z = acc + b_ref[...]
    ss = jnp.sum(z * z, axis=1, keepdims=True)
    inv = lax.rsqrt(jnp.maximum(ss, _EPS2))
    return _silu(z * inv * g_ref[...])


def _make_block_body(ho, wo, ci, co, pad_edges):
    wo2 = wo // 2

    def body(y_ref, wp_ref, wh_ref, b_ref, g_ref, o_ref):
        r = _conv_norm_silu(y_ref, wp_ref, wh_ref, b_ref, g_ref,
                            ho, wo, ci, co, pad_edges)
        o_ref[...] = jnp.zeros(o_ref.shape, o_ref.dtype)
        o_ref[1:ho + 1, 1:wo2 + 1, :] = (
            r.reshape(ho, wo2, 2 * co).astype(o_ref.dtype))
    return body


def _make_block2_body(ho, wo, ci, co, pad_edges):
    def body(y_ref, wp_ref, wh_ref, b_ref, g_ref, w1_ref, b1_ref, o_ref):
        r = _conv_norm_silu(y_ref, wp_ref, wh_ref, b_ref, g_ref,
                            ho, wo, ci, co, pad_edges)
        z = jnp.dot(r.astype(w1_ref.dtype), w1_ref[...],
                    preferred_element_type=jnp.float32) + b1_ref[...]
        o_ref[...] = _silu(z).astype(o_ref.dtype)
    return body


def _block(y, w, b, g, fuse1x1=None):
    n, hcells, _, wc, c2 = y.shape
    ho, wo, ci = hcells - 1, wc - 2, c2 // 2
    co = w.shape[0]
    m = ho * wo
    pad_edges = ci < 128

    wT = w.transpose(2, 3, 1, 0).astype(jnp.bfloat16)
    wp = jnp.stack([jnp.concatenate([wT[kh, 1], wT[kh, 2]], axis=0)
                    for kh in range(4)])
    zpad = jnp.zeros((ci, co), jnp.bfloat16)
    wh = []
    for kh in range(4):
        if pad_edges:
            wh.append(jnp.concatenate([zpad, wT[kh, 0]], axis=0))
            wh.append(jnp.concatenate([wT[kh, 3], zpad], axis=0))
        else:
            wh.append(wT[kh, 0])
            wh.append(wT[kh, 3])
    wh = jnp.stack(wh)
    kdim = wh.shape[1]
    bb = b.astype(jnp.float32).reshape(1, co)
    gg = ((float(co) ** 0.5) * (g.astype(jnp.float32) + 1.0)).reshape(1, co)

    in_specs = [
        pl.BlockSpec((None, hcells, 2, wc, c2), lambda i: (i, 0, 0, 0, 0)),
        pl.BlockSpec((4, 2 * ci, co), lambda i: (0, 0, 0)),
        pl.BlockSpec((8, kdim, co), lambda i: (0, 0, 0)),
        pl.BlockSpec((1, co), lambda i: (0, 0)),
        pl.BlockSpec((1, co), lambda i: (0, 0)),
    ]
    args = [y, wp, wh, bb, gg]
    flops = 2 * n * m * (16 + (8 if pad_edges else 0)) * ci * co
    if fuse1x1 is None:
        body = _make_block_body(ho, wo, ci, co, pad_edges)
        out_shape = jax.ShapeDtypeStruct((n, ho + 2, wo // 2 + 2, 2 * co),
                                         jnp.bfloat16)
        out_spec = pl.BlockSpec((None, ho + 2, wo // 2 + 2, 2 * co),
                                lambda i: (i, 0, 0, 0))
    else:
        w1, b1 = fuse1x1
        w1m = w1.reshape(co, co).T.astype(jnp.bfloat16)
        in_specs += [
            pl.BlockSpec((co, co), lambda i: (0, 0)),
            pl.BlockSpec((1, co), lambda i: (0, 0)),
        ]
        args += [w1m, b1.astype(jnp.float32).reshape(1, co)]
        flops += 2 * n * m * co * co
        body = _make_block2_body(ho, wo, ci, co, pad_edges)
        out_shape = jax.ShapeDtypeStruct((n, m, co), jnp.bfloat16)
        out_spec = pl.BlockSpec((None, m, co), lambda i: (i, 0, 0))

    return pl.pallas_call(
        body,
        out_shape=out_shape,
        grid=(n,),
        in_specs=in_specs,
        out_specs=out_spec,
        compiler_params=pltpu.CompilerParams(
            dimension_semantics=("parallel",),
        ),
        cost_estimate=pl.CostEstimate(
            flops=flops,
            transcendentals=2 * n * m * co,
            bytes_accessed=(y.size + n * m * co) * 2,
        ),
    )(*args)


def _make_layer0_body(hh, ww):
    def body(y_ref, w_ref, b_ref, o_ref):
        def row(h, carry):
            xs = y_ref[pl.ds(h, 5)].reshape(80, ww)
            z = jnp.dot(w_ref[...], xs,
                        preferred_element_type=jnp.float32) + b_ref[...]
            o_ref[h] = _silu(z).astype(o_ref.dtype)
            return carry
        lax.fori_loop(0, hh, row, 0)
    return body


def _layer0(x, w, b):
    n, _, hh, ww = x.shape
    co = w.shape[0]
    xp = jnp.pad(x.astype(jnp.bfloat16), ((0, 0), (0, 0), (2, 2), (2, 2)))
    sl = jnp.stack([xp[:, :, :, kw:kw + ww] for kw in range(5)], axis=1)
    sl = sl.transpose(0, 3, 1, 2, 4).reshape(n, hh + 4, 15, ww)
    sl = jnp.pad(sl, ((0, 0), (0, 0), (0, 1), (0, 0)))
    wt = w.transpose(0, 2, 3, 1).reshape(co, 5, 15)
    wt = jnp.pad(wt, ((0, 0), (0, 0), (0, 1))).reshape(co, 80)
    return pl.pallas_call(
        _make_layer0_body(hh, ww),
        out_shape=jax.ShapeDtypeStruct((n, hh, co, ww), jnp.bfloat16),
        grid=(n,),
        in_specs=[
            pl.BlockSpec((None, hh + 4, 16, ww), lambda i: (i, 0, 0, 0)),
            pl.BlockSpec((co, 80), lambda i: (0, 0)),
            pl.BlockSpec((co, 1), lambda i: (0, 0)),
        ],
        out_specs=pl.BlockSpec((None, hh, co, ww), lambda i: (i, 0, 0, 0)),
        compiler_params=pltpu.CompilerParams(
            dimension_semantics=("parallel",),
        ),
        cost_estimate=pl.CostEstimate(
            flops=2 * n * hh * ww * co * 75,
            transcendentals=2 * n * hh * ww * co,
            bytes_accessed=(n * hh * ww * (16 + co)) * 2,
        ),
    )(sl.astype(jnp.bfloat16), wt.astype(jnp.bfloat16),
      b.astype(jnp.float32).reshape(co, 1))


def kernel(layer0_w, layer0_b, block0_w, block0_b, block0_g,
           block1_w, block1_b, block1_g, block2_w, block2_b, block2_g,
           logits_w1, logits_b1, logits_w2, logits_b2, x):
    n, _, hh, ww = x.shape
    y0 = _layer0(x, layer0_w, layer0_b)
    c = y0.shape[2]
    y0 = jnp.pad(y0, ((0, 0), (1, 1), (0, 0), (2, 2)))
    y0 = y0.transpose(0, 1, 3, 2)
    y0 = y0.reshape(n, hh // 2 + 1, 2, ww // 2 + 2, 2 * c)

    h = _block(y0, block0_w, block0_b, block0_g)
    h = h.reshape(n, h.shape[1] // 2, 2, h.shape[2], h.shape[3])
    h = _block(h, block1_w, block1_b, block1_g)
    h = h.reshape(n, h.shape[1] // 2, 2, h.shape[2], h.shape[3])
    h = _block(h, block2_w, block2_b, block2_g,
               fuse1x1=(logits_w1, logits_b1))
    ho, wo = hh // 8, ww // 8
    h = h.reshape(n, ho, wo, h.shape[-1])

    preds = lax.conv_general_dilated(
        h, logits_w2.astype(jnp.bfloat16),
        window_strides=(1, 1), padding="VALID",
        dimension_numbers=("NHWC", "OIHW", "NCHW"),
        preferred_element_type=jnp.float32)
    return preds + logits_b2.reshape(1, -1, 1, 1)


# speedup vs baseline: 1.5167x; 1.5167x over previous
import jax
import jax.numpy as jnp
from jax import lax
from jax.experimental import pallas as pl
from jax.experimental.pallas import tpu as pltpu

_EPS2 = 1e-24


def _silu(y):
    return y * jax.nn.sigmoid(y)


def _conv_norm_silu(y_ref, wp_ref, wh_ref, b_ref, g_ref, ho, wo, ci, co,
                    pad_edges):
    m = ho * wo
    c2 = 2 * ci
    acc = jnp.zeros((m, co), jnp.float32)
    for kh in range(4):
        base, par = kh // 2, kh % 2
        xs = y_ref[base:base + ho, par, 1:1 + wo, :].reshape(m, c2)
        acc = acc + jnp.dot(xs, wp_ref[kh],
                            preferred_element_type=jnp.float32)
        if pad_edges:
            x0 = y_ref[base:base + ho, par, 0:wo, :].reshape(m, c2)
            x2 = y_ref[base:base + ho, par, 2:2 + wo, :].reshape(m, c2)
        else:
            x0 = y_ref[base:base + ho, par, 0:wo, ci:].reshape(m, ci)
            x2 = y_ref[base:base + ho, par, 2:2 + wo, :ci].reshape(m, ci)
        acc = acc + jnp.dot(x0, wh_ref[2 * kh],
                            preferred_element_type=jnp.float32)
        acc = acc + jnp.dot(x2, wh_ref[2 * kh + 1],
                            preferred_element_type=jnp.float32)
    z = acc + b_ref[...]
    ss = jnp.sum(z * z, axis=1, keepdims=True)
    inv = lax.rsqrt(jnp.maximum(ss, _EPS2))
    return _silu(z * inv * g_ref[...])


def _make_block_body(ho, wo, ci, co, pad_edges):
    wo2 = wo // 2

    def body(y_ref, wp_ref, wh_ref, b_ref, g_ref, o_ref):
        r = _conv_norm_silu(y_ref, wp_ref, wh_ref, b_ref, g_ref,
                            ho, wo, ci, co, pad_edges)
        o_ref[...] = jnp.zeros(o_ref.shape, o_ref.dtype)
        o_ref[1:ho + 1, 1:wo2 + 1, :] = (
            r.reshape(ho, wo2, 2 * co).astype(o_ref.dtype))
    return body


def _make_block2_body(ho, wo, ci, co, pad_edges):
    def body(y_ref, wp_ref, wh_ref, b_ref, g_ref, w1_ref, b1_ref, o_ref):
        r = _conv_norm_silu(y_ref, wp_ref, wh_ref, b_ref, g_ref,
                            ho, wo, ci, co, pad_edges)
        z = jnp.dot(r.astype(w1_ref.dtype), w1_ref[...],
                    preferred_element_type=jnp.float32) + b1_ref[...]
        o_ref[...] = _silu(z).astype(o_ref.dtype)
    return body


def _block(y, w, b, g, fuse1x1=None):
    n, hcells, _, wc, c2 = y.shape
    ho, wo, ci = hcells - 1, wc - 2, c2 // 2
    co = w.shape[0]
    m = ho * wo
    pad_edges = ci < 128

    wT = w.transpose(2, 3, 1, 0).astype(jnp.bfloat16)
    wp = jnp.stack([jnp.concatenate([wT[kh, 1], wT[kh, 2]], axis=0)
                    for kh in range(4)])
    zpad = jnp.zeros((ci, co), jnp.bfloat16)
    wh = []
    for kh in range(4):
        if pad_edges:
            wh.append(jnp.concatenate([zpad, wT[kh, 0]], axis=0))
            wh.append(jnp.concatenate([wT[kh, 3], zpad], axis=0))
        else:
            wh.append(wT[kh, 0])
            wh.append(wT[kh, 3])
    wh = jnp.stack(wh)
    kdim = wh.shape[1]
    bb = b.astype(jnp.float32).reshape(1, co)
    gg = ((float(co) ** 0.5) * (g.astype(jnp.float32) + 1.0)).reshape(1, co)

    in_specs = [
        pl.BlockSpec((None, hcells, 2, wc, c2), lambda i: (i, 0, 0, 0, 0)),
        pl.BlockSpec((4, 2 * ci, co), lambda i: (0, 0, 0)),
        pl.BlockSpec((8, kdim, co), lambda i: (0, 0, 0)),
        pl.BlockSpec((1, co), lambda i: (0, 0)),
        pl.BlockSpec((1, co), lambda i: (0, 0)),
    ]
    args = [y, wp, wh, bb, gg]
    flops = 2 * n * m * (16 + (8 if pad_edges else 0)) * ci * co
    if fuse1x1 is None:
        body = _make_block_body(ho, wo, ci, co, pad_edges)
        out_shape = jax.ShapeDtypeStruct((n, ho + 2, wo // 2 + 2, 2 * co),
                                         jnp.bfloat16)
        out_spec = pl.BlockSpec((None, ho + 2, wo // 2 + 2, 2 * co),
                                lambda i: (i, 0, 0, 0))
    else:
        w1, b1 = fuse1x1
        w1m = w1.reshape(co, co).T.astype(jnp.bfloat16)
        in_specs += [
            pl.BlockSpec((co, co), lambda i: (0, 0)),
            pl.BlockSpec((1, co), lambda i: (0, 0)),
        ]
        args += [w1m, b1.astype(jnp.float32).reshape(1, co)]
        flops += 2 * n * m * co * co
        body = _make_block2_body(ho, wo, ci, co, pad_edges)
        out_shape = jax.ShapeDtypeStruct((n, m, co), jnp.bfloat16)
        out_spec = pl.BlockSpec((None, m, co), lambda i: (i, 0, 0))

    return pl.pallas_call(
        body,
        out_shape=out_shape,
        grid=(n,),
        in_specs=in_specs,
        out_specs=out_spec,
        compiler_params=pltpu.CompilerParams(
            dimension_semantics=("parallel",),
        ),
        cost_estimate=pl.CostEstimate(
            flops=flops,
            transcendentals=2 * n * m * co,
            bytes_accessed=(y.size + n * m * co) * 2,
        ),
    )(*args)


def _make_layer0_body(hh, ww):
    def body(y_ref, w_ref, b_ref, o_ref):
        def strip(s, carry):
            base = s * 8
            for r in range(8):
                xs = y_ref[pl.ds(base + r, 5)].reshape(80, ww)
                z = jnp.dot(w_ref[...], xs,
                            preferred_element_type=jnp.float32) + b_ref[...]
                o_ref[base + r] = _silu(z).astype(o_ref.dtype)
            return carry
        lax.fori_loop(0, hh // 8, strip, 0)
    return body


def _layer0(x, w, b):
    n, _, hh, ww = x.shape
    co = w.shape[0]
    xp = jnp.pad(x.astype(jnp.bfloat16), ((0, 0), (0, 0), (2, 2), (2, 2)))
    sl = jnp.stack([xp[:, :, :, kw:kw + ww] for kw in range(5)], axis=1)
    sl = sl.transpose(0, 3, 1, 2, 4).reshape(n, hh + 4, 15, ww)
    sl = jnp.pad(sl, ((0, 0), (0, 0), (0, 1), (0, 0)))
    wt = w.transpose(0, 2, 3, 1).reshape(co, 5, 15)
    wt = jnp.pad(wt, ((0, 0), (0, 0), (0, 1))).reshape(co, 80)
    return pl.pallas_call(
        _make_layer0_body(hh, ww),
        out_shape=jax.ShapeDtypeStruct((n, hh, co, ww), jnp.bfloat16),
        grid=(n,),
        in_specs=[
            pl.BlockSpec((None, hh + 4, 16, ww), lambda i: (i, 0, 0, 0)),
            pl.BlockSpec((co, 80), lambda i: (0, 0)),
            pl.BlockSpec((co, 1), lambda i: (0, 0)),
        ],
        out_specs=pl.BlockSpec((None, hh, co, ww), lambda i: (i, 0, 0, 0)),
        compiler_params=pltpu.CompilerParams(
            dimension_semantics=("parallel",),
        ),
        cost_estimate=pl.CostEstimate(
            flops=2 * n * hh * ww * co * 75,
            transcendentals=2 * n * hh * ww * co,
            bytes_accessed=(n * hh * ww * (16 + co)) * 2,
        ),
    )(sl.astype(jnp.bfloat16), wt.astype(jnp.bfloat16),
      b.astype(jnp.float32).reshape(co, 1))


def kernel(layer0_w, layer0_b, block0_w, block0_b, block0_g,
           block1_w, block1_b, block1_g, block2_w, block2_b, block2_g,
           logits_w1, logits_b1, logits_w2, logits_b2, x):
    n, _, hh, ww = x.shape
    y0 = _layer0(x, layer0_w, layer0_b)
    c = y0.shape[2]
    y0 = jnp.pad(y0, ((0, 0), (1, 1), (0, 0), (2, 2)))
    y0 = y0.transpose(0, 1, 3, 2)
    y0 = y0.reshape(n, hh // 2 + 1, 2, ww // 2 + 2, 2 * c)

    h = _block(y0, block0_w, block0_b, block0_g)
    h = h.reshape(n, h.shape[1] // 2, 2, h.shape[2], h.shape[3])
    h = _block(h, block1_w, block1_b, block1_g)
    h = h.reshape(n, h.shape[1] // 2, 2, h.shape[2], h.shape[3])
    h = _block(h, block2_w, block2_b, block2_g,
               fuse1x1=(logits_w1, logits_b1))
    ho, wo = hh // 8, ww // 8
    h = h.reshape(n, ho, wo, h.shape[-1])

    preds = lax.conv_general_dilated(
        h, logits_w2.astype(jnp.bfloat16),
        window_strides=(1, 1), padding="VALID",
        dimension_numbers=("NHWC", "OIHW", "NCHW"),
        preferred_element_type=jnp.float32)
    return preds + logits_b2.reshape(1, -1, 1, 1)
